# baseline (speedup 1.0000x reference)
import functools

import jax
import jax.numpy as jnp
from jax import lax
from jax.experimental import pallas as pl
from jax.experimental.pallas import tpu as pltpu
from jax.experimental.pallas import tpu_sc as plsc

_B, _K, _D, _H, _W = 64, 5, 768, 32, 32
_N = _H * _W
_NW = 32
_PT = (_B * _K) // _NW
_R = _PT * 9
_RP = 96


_BB = 8


def _sims_argmax_body(cue_ref, patches_ref, idx_ref):
    for bi in range(_BB):
        p = patches_ref[bi].reshape(_N, _D)
        c = cue_ref[bi]
        norm2 = jnp.sum(p * p, axis=1, keepdims=True)
        inv = 1.0 / jnp.maximum(jnp.sqrt(norm2), 1e-12)
        cn2 = jnp.sum(c * c, axis=1, keepdims=True)
        cinv = 1.0 / jnp.maximum(jnp.sqrt(cn2), 1e-12)
        pn = (p * inv).astype(jnp.bfloat16)
        cn = (c * cinv).astype(jnp.bfloat16)
        dn = (((1,), (1,)), ((), ()))
        s = lax.dot_general(pn, cn, dn, preferred_element_type=jnp.float32)
        m = jnp.max(s, axis=0, keepdims=True)
        row = lax.broadcasted_iota(jnp.int32, (_N, _K), 0)
        cand = jnp.where(s == m, row, _N)
        idxs = jnp.min(cand, axis=0, keepdims=True)
        idx_ref[bi] = jnp.concatenate(
            [idxs, jnp.zeros((1, 8 - _K), jnp.int32)], axis=1)


def _sims_argmax(cue, patches):
    return pl.pallas_call(
        _sims_argmax_body,
        grid=(_B // _BB,),
        in_specs=[
            pl.BlockSpec((_BB, _K, _D), lambda b: (b, 0, 0)),
            pl.BlockSpec((_BB, _H, _W, _D), lambda b: (b, 0, 0, 0)),
        ],
        out_specs=pl.BlockSpec((_BB, 1, 8), lambda b: (b, 0, 0)),
        out_shape=jax.ShapeDtypeStruct((_B, 1, 8), jnp.int32),
    )(cue, patches)


def _gather_avg(table, gidx, gw):
    mesh = plsc.VectorSubcoreMesh(core_axis_name="c", subcore_axis_name="s")

    @functools.partial(
        pl.kernel,
        out_type=jax.ShapeDtypeStruct((_B, _K, _D), jnp.float32),
        mesh=mesh,
        scratch_types=[
            pltpu.VMEM((_RP,), jnp.int32),
            pltpu.VMEM((_RP, _D), jnp.float32),
            pltpu.VMEM((_RP, 16), jnp.float32),
            pltpu.VMEM((2, _K, _D), jnp.float32),
            pltpu.SemaphoreType.DMA,
            pltpu.SemaphoreType.DMA,
        ],
    )
    def sc_kernel(gidx_hbm, gw_hbm, table_hbm, out_hbm,
                  idx_v, rows_v, w_v, acc_v, sem0, sem1):
        wid = lax.axis_index("s") * 2 + lax.axis_index("c")
        pltpu.sync_copy(gidx_hbm.at[wid, 0], idx_v)
        half = _RP // 2
        cp0 = pltpu.async_copy(
            table_hbm.at[idx_v.at[pl.ds(0, half)]],
            rows_v.at[pl.ds(0, half)], sem0)
        cp1 = pltpu.async_copy(
            table_hbm.at[idx_v.at[pl.ds(half, half)]],
            rows_v.at[pl.ds(half, half)], sem1)
        pltpu.sync_copy(gw_hbm.at[wid], w_v)

        def do_pair(j):
            wj = [w_v[j * 9 + r, :] for r in range(9)]

            @plsc.parallel_loop(0, _D // 16, unroll=4)
            def chunk(cidx, _j=j, _w=wj):
                off = cidx * 16
                acc = rows_v[_j * 9, pl.ds(off, 16)] * _w[0]
                for r in range(1, 9):
                    acc = acc + rows_v[_j * 9 + r, pl.ds(off, 16)] * _w[r]
                acc_v[_j // _K, _j % _K, pl.ds(off, 16)] = acc

        cp0.wait()
        for j in range(_K):
            do_pair(j)
        cp1.wait()
        pltpu.sync_copy(acc_v.at[0], out_hbm.at[2 * wid])
        for j in range(_K, _PT):
            do_pair(j)
        pltpu.sync_copy(acc_v.at[1], out_hbm.at[2 * wid + 1])

    return sc_kernel(gidx, gw, table)


def kernel(cue, patches):
    idx3 = _sims_argmax(cue, patches)
    idx = idx3[:, 0, :_K]
    y, x = idx // _W, idx % _W
    dy = jnp.array([-1, -1, -1, 0, 0, 0, 1, 1, 1], jnp.int32)
    dx = jnp.array([-1, 0, 1, -1, 0, 1, -1, 0, 1], jnp.int32)
    yy = y[..., None] + dy
    xx = x[..., None] + dx
    valid = (yy >= 0) & (yy < _H) & (xx >= 0) & (xx < _W)
    flat = (jnp.arange(_B, dtype=jnp.int32)[:, None, None] * _N
            + jnp.clip(yy, 0, _H - 1) * _W + jnp.clip(xx, 0, _W - 1))
    w = jnp.where(valid, jnp.float32(1.0 / 9.0), jnp.float32(0.0))
    gidx = flat.reshape(_NW, _R).astype(jnp.int32)
    gidx = jnp.pad(gidx, ((0, 0), (0, _RP - _R))).reshape(_NW, 1, _RP)
    gw = jnp.pad(w.reshape(_NW, _R), ((0, 0), (0, _RP - _R)))
    gw = jnp.broadcast_to(gw[..., None], (_NW, _RP, 16))
    table = patches.reshape(_B * _N, _D)
    return _gather_avg(table, gidx, gw)


# speedup vs baseline: 1.0072x; 1.0072x over previous
import functools

import jax
import jax.numpy as jnp
from jax import lax
from jax.experimental import pallas as pl
from jax.experimental.pallas import tpu as pltpu
from jax.experimental.pallas import tpu_sc as plsc

_B, _K, _D, _H, _W = 64, 5, 768, 32, 32
_N = _H * _W
_NW = 32
_PT = (_B * _K) // _NW
_R = _PT * 9
_RP = 96


_BB = 4


def _sims_argmax_body(cue_ref, patches_ref, idx_ref):
    for bi in range(_BB):
        p = patches_ref[bi].reshape(_N, _D)
        c = cue_ref[bi]
        norm2 = jnp.sum(p * p, axis=1, keepdims=True)
        inv = 1.0 / jnp.maximum(jnp.sqrt(norm2), 1e-12)
        cn2 = jnp.sum(c * c, axis=1, keepdims=True)
        cinv = 1.0 / jnp.maximum(jnp.sqrt(cn2), 1e-12)
        pn = (p * inv).astype(jnp.bfloat16)
        cn = (c * cinv).astype(jnp.bfloat16)
        dn = (((1,), (1,)), ((), ()))
        s = lax.dot_general(pn, cn, dn, preferred_element_type=jnp.float32)
        m = jnp.max(s, axis=0, keepdims=True)
        row = lax.broadcasted_iota(jnp.int32, (_N, _K), 0)
        cand = jnp.where(s == m, row, _N)
        idxs = jnp.min(cand, axis=0, keepdims=True)
        idx_ref[bi] = jnp.concatenate(
            [idxs, jnp.zeros((1, 8 - _K), jnp.int32)], axis=1)


def _sims_argmax(cue, patches):
    return pl.pallas_call(
        _sims_argmax_body,
        grid=(_B // _BB,),
        in_specs=[
            pl.BlockSpec((_BB, _K, _D), lambda b: (b, 0, 0)),
            pl.BlockSpec((_BB, _H, _W, _D), lambda b: (b, 0, 0, 0)),
        ],
        out_specs=pl.BlockSpec((_BB, 1, 8), lambda b: (b, 0, 0)),
        out_shape=jax.ShapeDtypeStruct((_B, 1, 8), jnp.int32),
    )(cue, patches)


def _gather_avg(table, gidx, gw):
    mesh = plsc.VectorSubcoreMesh(core_axis_name="c", subcore_axis_name="s")

    @functools.partial(
        pl.kernel,
        out_type=jax.ShapeDtypeStruct((_B, _K, _D), jnp.float32),
        mesh=mesh,
        scratch_types=[
            pltpu.VMEM((_RP,), jnp.int32),
            pltpu.VMEM((_RP, _D), jnp.float32),
            pltpu.VMEM((_RP, 16), jnp.float32),
            pltpu.VMEM((2, _K, _D), jnp.float32),
            pltpu.SemaphoreType.DMA,
            pltpu.SemaphoreType.DMA,
        ],
    )
    def sc_kernel(gidx_hbm, gw_hbm, table_hbm, out_hbm,
                  idx_v, rows_v, w_v, acc_v, sem0, sem1):
        wid = lax.axis_index("s") * 2 + lax.axis_index("c")
        pltpu.sync_copy(gidx_hbm.at[wid, 0], idx_v)
        half = _RP // 2
        cp0 = pltpu.async_copy(
            table_hbm.at[idx_v.at[pl.ds(0, half)]],
            rows_v.at[pl.ds(0, half)], sem0)
        cp1 = pltpu.async_copy(
            table_hbm.at[idx_v.at[pl.ds(half, half)]],
            rows_v.at[pl.ds(half, half)], sem1)
        pltpu.sync_copy(gw_hbm.at[wid], w_v)

        def do_pair(j):
            wj = [w_v[j * 9 + r, :] for r in range(9)]

            @plsc.parallel_loop(0, _D // 16, unroll=4)
            def chunk(cidx, _j=j, _w=wj):
                off = cidx * 16
                acc = rows_v[_j * 9, pl.ds(off, 16)] * _w[0]
                for r in range(1, 9):
                    acc = acc + rows_v[_j * 9 + r, pl.ds(off, 16)] * _w[r]
                acc_v[_j // _K, _j % _K, pl.ds(off, 16)] = acc

        cp0.wait()
        for j in range(_K):
            do_pair(j)
        cp1.wait()
        pltpu.sync_copy(acc_v.at[0], out_hbm.at[2 * wid])
        for j in range(_K, _PT):
            do_pair(j)
        pltpu.sync_copy(acc_v.at[1], out_hbm.at[2 * wid + 1])

    return sc_kernel(gidx, gw, table)


def kernel(cue, patches):
    idx3 = _sims_argmax(cue, patches)
    idx = idx3[:, 0, :_K]
    y, x = idx // _W, idx % _W
    dy = jnp.array([-1, -1, -1, 0, 0, 0, 1, 1, 1], jnp.int32)
    dx = jnp.array([-1, 0, 1, -1, 0, 1, -1, 0, 1], jnp.int32)
    yy = y[..., None] + dy
    xx = x[..., None] + dx
    valid = (yy >= 0) & (yy < _H) & (xx >= 0) & (xx < _W)
    flat = (jnp.arange(_B, dtype=jnp.int32)[:, None, None] * _N
            + jnp.clip(yy, 0, _H - 1) * _W + jnp.clip(xx, 0, _W - 1))
    w = jnp.where(valid, jnp.float32(1.0 / 9.0), jnp.float32(0.0))
    gidx = flat.reshape(_NW, _R).astype(jnp.int32)
    gidx = jnp.pad(gidx, ((0, 0), (0, _RP - _R))).reshape(_NW, 1, _RP)
    gw = jnp.pad(w.reshape(_NW, _R), ((0, 0), (0, _RP - _R)))
    gw = jnp.broadcast_to(gw[..., None], (_NW, _RP, 16))
    table = patches.reshape(_B * _N, _D)
    return _gather_avg(table, gidx, gw)
